# baseline (speedup 1.0000x reference)
import functools

import jax
import jax.numpy as jnp
from jax import lax
from jax.experimental import pallas as pl
from jax.experimental.pallas import tpu as pltpu
from jax.experimental.pallas import tpu_sc as plsc

P = 5
C = 64
CH = 4 * C
HH = 160
WW = 160
NB = 2
T = 2
LL = (HH // P) * (WW // P)
D = P * P * C
DP = 1664

NW = 32
ROWS_TOTAL = T * NB * 3 * LL
ROWS_PER_W = ROWS_TOTAL // NW
CHUNK = 64


def _sc_gather(table, fidx):
    mesh = plsc.VectorSubcoreMesh(core_axis_name="c", subcore_axis_name="s")

    @functools.partial(
        pl.kernel,
        mesh=mesh,
        out_type=jax.ShapeDtypeStruct((ROWS_TOTAL, DP), jnp.float32),
        scratch_types=[
            pltpu.VMEM((CHUNK,), jnp.int32),
            pltpu.VMEM((CHUNK, DP), jnp.float32),
            pltpu.SemaphoreType.DMA,
        ],
    )
    def k(table_hbm, idx_hbm, out_hbm, idx_v, rows_v, sem):
        wid = lax.axis_index("s") * 2 + lax.axis_index("c")
        base = wid * ROWS_PER_W
        for ci in range(ROWS_PER_W // CHUNK):
            off = base + ci * CHUNK
            pltpu.sync_copy(idx_hbm.at[pl.ds(off, CHUNK)], idx_v)
            pltpu.async_copy(table_hbm.at[idx_v], rows_v, sem).wait()
            pltpu.sync_copy(rows_v, out_hbm.at[pl.ds(off, CHUNK)])

    return k(table, fidx)


BR = 16


def _conv_body(v0, v1, v2, w_ref, b_ref, res_ref, out_ref):
    acc = jnp.zeros((BR * WW, C), jnp.float32)
    for dy, v in enumerate((v0, v1, v2)):
        x = v[0]
        for dx in range(3):
            xs = x[:, dx:dx + WW, :].reshape(BR * WW, CH)
            acc += jnp.dot(xs, w_ref[0, dy, dx], preferred_element_type=jnp.float32)
    acc = acc + b_ref[0]
    acc = acc + res_ref[0].reshape(BR * WW, C)
    out_ref[0] = acc.reshape(BR, WW, C)


def _conv_specs():
    ix = lambda i, r: (i, r, 0, 0)
    in_specs = [
        pl.BlockSpec((1, BR, WW + 2, CH), ix),
        pl.BlockSpec((1, BR, WW + 2, CH), ix),
        pl.BlockSpec((1, BR, WW + 2, CH), ix),
        pl.BlockSpec((1, 3, 3, CH, C), lambda i, r: (i // 2, 0, 0, 0, 0)),
        pl.BlockSpec((1, 1, C), lambda i, r: (i // 2, 0, 0)),
        pl.BlockSpec((1, BR, WW, C), ix),
    ]
    out_spec = pl.BlockSpec((1, BR, WW, C), ix)
    return in_specs, out_spec


def _conv(v0, v1, v2, W_all, b_all, res_all):
    in_specs, out_spec = _conv_specs()
    return pl.pallas_call(
        _conv_body,
        grid=(T * NB, HH // BR),
        in_specs=in_specs,
        out_specs=out_spec,
        out_shape=jax.ShapeDtypeStruct((T * NB, HH, WW, C), jnp.float32),
    )(v0, v1, v2, W_all, b_all, res_all)


def _prep_w(Wx, gx):
    w = Wx.transpose(2, 3, 1, 0)
    return w * jnp.repeat(gx, C)[None, None, :, None]


def kernel(x_rgb, y_hsi, corr, rgb_gamma, hsi_gamma, Wr, br, Wh, bh):
    half = C // 2
    rgb = jnp.concatenate([x_rgb[:, half:], y_hsi[:, half:]], axis=1)
    hsi = jnp.concatenate([x_rgb[:, :half], y_hsi[:, :half]], axis=1)
    base_cl = jnp.stack([rgb, hsi]).transpose(0, 1, 3, 4, 2)

    table = (
        base_cl.reshape(T, NB, HH // P, P, WW // P, P, C)
        .transpose(0, 1, 2, 4, 3, 5, 6)
        .reshape(T * NB * LL, D)
    )
    table = jnp.pad(table, ((0, 0), (0, DP - D)))
    idx = corr[0]
    idx3 = jnp.transpose(idx[:, :, 1:4], (0, 2, 1))
    fidx = (
        jnp.arange(T * NB, dtype=jnp.int32).reshape(T, NB, 1, 1) * LL
        + idx3[None].astype(jnp.int32)
    ).reshape(ROWS_TOTAL)

    g = _sc_gather(table, fidx)[:, :D]
    g = (
        g.reshape(T, NB, 3, HH // P, WW // P, P, P, C)
        .transpose(0, 1, 3, 5, 4, 6, 2, 7)
        .reshape(T, NB, HH, WW, 3 * C)
    )
    img = jnp.concatenate([base_cl, g], axis=-1).astype(jnp.bfloat16)
    padded = jnp.pad(img, ((0, 0), (0, 0), (1, 1), (1, 1), (0, 0))).reshape(
        T * NB, HH + 2, WW + 2, CH
    )
    v0 = padded[:, 0:HH]
    v1 = padded[:, 1:HH + 1]
    v2 = padded[:, 2:HH + 2]

    W_all = jnp.stack([_prep_w(Wr, rgb_gamma), _prep_w(Wh, hsi_gamma)]).astype(
        jnp.bfloat16
    )
    b_all = jnp.stack([br, bh]).reshape(T, 1, C)
    res_all = base_cl.reshape(T * NB, HH, WW, C)

    out = _conv(v0, v1, v2, W_all, b_all, res_all)
    out = out.reshape(T, NB, HH, WW, C).transpose(0, 1, 4, 2, 3)
    return (out[0], out[1])


# speedup vs baseline: 1.3198x; 1.3198x over previous
import functools

import jax
import jax.numpy as jnp
from jax import lax
from jax.experimental import pallas as pl
from jax.experimental.pallas import tpu as pltpu
from jax.experimental.pallas import tpu_sc as plsc

P = 5
C = 64
CH = 4 * C
HH = 160
WW = 160
NB = 2
T = 2
LL = (HH // P) * (WW // P)
D = P * P * C
DP = 1664

NW = 32
ROWS_TOTAL = T * NB * 3 * LL
ROWS_PER_W = ROWS_TOTAL // NW
CHUNK = 64


def _sc_gather(table, fidx):
    mesh = plsc.VectorSubcoreMesh(core_axis_name="c", subcore_axis_name="s")

    @functools.partial(
        pl.kernel,
        mesh=mesh,
        out_type=jax.ShapeDtypeStruct((ROWS_TOTAL, DP), jnp.float32),
        scratch_types=[
            pltpu.VMEM((CHUNK,), jnp.int32),
            pltpu.VMEM((CHUNK, DP), jnp.float32),
            pltpu.SemaphoreType.DMA,
        ],
    )
    def k(table_hbm, idx_hbm, out_hbm, idx_v, rows_v, sem):
        wid = lax.axis_index("s") * 2 + lax.axis_index("c")
        base = wid * ROWS_PER_W
        for ci in range(ROWS_PER_W // CHUNK):
            off = base + ci * CHUNK
            pltpu.sync_copy(idx_hbm.at[pl.ds(off, CHUNK)], idx_v)
            pltpu.async_copy(table_hbm.at[idx_v], rows_v, sem).wait()
            pltpu.sync_copy(rows_v, out_hbm.at[pl.ds(off, CHUNK)])

    return k(table, fidx)


BR = 16
NRB = HH // BR


def _conv_body(pad_hbm, w_ref, b_ref, out_ref, buf, sems):
    i = pl.program_id(0)
    r = pl.program_id(1)
    k = i * NRB + r
    slot = lax.rem(k, 2)

    def copy_in(ii, rr, s):
        return pltpu.make_async_copy(
            pad_hbm.at[ii, pl.ds(rr * BR, BR + 2)], buf.at[s], sems.at[s]
        )

    @pl.when(k == 0)
    def _():
        copy_in(i, r, slot).start()

    @pl.when(k < T * NB * NRB - 1)
    def _():
        kn = k + 1
        copy_in(kn // NRB, lax.rem(kn, NRB), 1 - slot).start()

    copy_in(i, r, slot).wait()
    x = buf[slot]
    acc = jnp.zeros((BR * WW, C), jnp.float32)
    for dy in range(3):
        xd = x[dy:dy + BR]
        for dx in range(3):
            xs = xd[:, dx:dx + WW, :].reshape(BR * WW, CH)
            acc += jnp.dot(xs, w_ref[0, dy, dx], preferred_element_type=jnp.float32)
    acc = acc + b_ref[0]
    acc = acc + x[1:1 + BR, 1:1 + WW, :C].reshape(BR * WW, C)
    out_ref[0] = acc.reshape(BR, WW, C)


def _conv(padded, W_all, b_all):
    return pl.pallas_call(
        _conv_body,
        grid=(T * NB, NRB),
        in_specs=[
            pl.BlockSpec(memory_space=pl.ANY),
            pl.BlockSpec((1, 3, 3, CH, C), lambda i, r: (i // 2, 0, 0, 0, 0)),
            pl.BlockSpec((1, 1, C), lambda i, r: (i // 2, 0, 0)),
        ],
        out_specs=pl.BlockSpec((1, BR, WW, C), lambda i, r: (i, r, 0, 0)),
        out_shape=jax.ShapeDtypeStruct((T * NB, HH, WW, C), jnp.float32),
        scratch_shapes=[
            pltpu.VMEM((2, BR + 2, WW + 2, CH), jnp.float32),
            pltpu.SemaphoreType.DMA((2,)),
        ],
    )(padded, W_all, b_all)


def _prep_w(Wx, gx):
    w = Wx.transpose(2, 3, 1, 0)
    return w * jnp.repeat(gx, C)[None, None, :, None]


def kernel(x_rgb, y_hsi, corr, rgb_gamma, hsi_gamma, Wr, br, Wh, bh):
    half = C // 2
    rgb = jnp.concatenate([x_rgb[:, half:], y_hsi[:, half:]], axis=1)
    hsi = jnp.concatenate([x_rgb[:, :half], y_hsi[:, :half]], axis=1)
    base_cl = jnp.stack([rgb, hsi]).transpose(0, 1, 3, 4, 2)

    table = (
        base_cl.reshape(T, NB, HH // P, P, WW // P, P, C)
        .transpose(0, 1, 2, 4, 3, 5, 6)
        .reshape(T * NB * LL, D)
    )
    table = jnp.pad(table, ((0, 0), (0, DP - D)))
    idx = corr[0]
    idx3 = jnp.transpose(idx[:, :, 1:4], (0, 2, 1))
    fidx = (
        jnp.arange(T * NB, dtype=jnp.int32).reshape(T, NB, 1, 1) * LL
        + idx3[None].astype(jnp.int32)
    ).reshape(ROWS_TOTAL)

    g = _sc_gather(table, fidx)[:, :D]
    g = (
        g.reshape(T, NB, 3, HH // P, WW // P, P, P, C)
        .transpose(0, 1, 3, 5, 4, 6, 2, 7)
        .reshape(T, NB, HH, WW, 3 * C)
    )
    img = jnp.concatenate([base_cl, g], axis=-1)
    padded = jnp.pad(img, ((0, 0), (0, 0), (1, 1), (1, 1), (0, 0))).reshape(
        T * NB, HH + 2, WW + 2, CH
    )

    W_all = jnp.stack([_prep_w(Wr, rgb_gamma), _prep_w(Wh, hsi_gamma)])
    b_all = jnp.stack([br, bh]).reshape(T, 1, C)

    out = _conv(padded, W_all, b_all)
    out = out.reshape(T, NB, HH, WW, C).transpose(0, 1, 4, 2, 3)
    return (out[0], out[1])
